# speedup vs baseline: 1.4448x; 1.4448x over previous
import jax
import jax.numpy as jnp
from jax.experimental import pallas as pl
from jax.experimental.pallas import tpu as pltpu

B, S, D = 4, 8192, 768
BS = 512


def _body(x_ref, pe_ref, o_ref):
    o_ref[0] = x_ref[0] + pe_ref[...]


def kernel(x, pe):
    grid = (S // BS, B)
    return pl.pallas_call(
        _body,
        grid=grid,
        in_specs=[
            pl.BlockSpec((1, BS, D), lambda s, b: (b, s, 0)),
            pl.BlockSpec((BS, D), lambda s, b: (s, 0)),
        ],
        out_specs=pl.BlockSpec((1, BS, D), lambda s, b: (b, s, 0)),
        out_shape=jax.ShapeDtypeStruct((B, S, D), x.dtype),
        compiler_params=pltpu.CompilerParams(
            dimension_semantics=("arbitrary", "arbitrary"),
        ),
    )(x, pe[:S])
